# baseline (speedup 1.0000x reference)
import functools

import jax
import jax.numpy as jnp
from jax import lax
from jax.experimental import pallas as pl
from jax.experimental.pallas import tpu as pltpu
from jax.experimental.pallas import tpu_sc as plsc

D = 64
HIST = 50
LANES = 128
HPAD = 56
G = 4
C = G * HPAD


@functools.lru_cache(maxsize=None)
def _make_gather(BATCH: int, NBUF: int):
    info = plsc.get_sparse_core_info()
    NC, NS = info.num_cores, info.num_subcores
    NW = NC * NS
    i_per_w = BATCH // NW
    n_chunks = i_per_w // G
    assert i_per_w * NW == BATCH and n_chunks * G == i_per_w
    assert n_chunks % NBUF == 0
    n_idx = i_per_w * HPAD

    mesh = plsc.VectorSubcoreMesh(core_axis_name="c", subcore_axis_name="s")

    @functools.partial(
        pl.kernel,
        mesh=mesh,
        compiler_params=pltpu.CompilerParams(use_tc_tiling_on_sc=False),
        out_type=jax.ShapeDtypeStruct((BATCH, HPAD, LANES), jnp.float32),
        scratch_types=[
            pltpu.VMEM((n_idx,), jnp.int32),
            pltpu.VMEM((NBUF, C, D), jnp.float32),
            pltpu.SemaphoreType.DMA((NBUF,)),
            pltpu.SemaphoreType.DMA((NBUF,)),
        ],
    )
    def gather_kernel(idx_hbm, table_hbm, out_hbm, idx_v, rows_v, sem_g, sem_o):
        wid = lax.axis_index("s") * NC + lax.axis_index("c")
        i_base = wid * i_per_w

        pltpu.sync_copy(idx_hbm.at[pl.ds(i_base * HPAD, n_idx)], idx_v)

        def gather(g, b):
            pltpu.async_copy(
                table_hbm.at[idx_v.at[pl.ds(g * C, C)]],
                rows_v.at[b],
                sem_g.at[b],
            )

        def wait_gather(g, b):
            pltpu.make_async_copy(
                table_hbm.at[idx_v.at[pl.ds(g * C, C)]],
                rows_v.at[b],
                sem_g.at[b],
            ).wait()

        def writeback(g, b):
            for t in range(G):
                pltpu.async_copy(
                    rows_v.at[b, pl.ds(t * HPAD, HPAD), :],
                    out_hbm.at[i_base + g * G + t, :, pl.ds(0, D)],
                    sem_o.at[b],
                )

        def wait_writeback(g, b):
            for t in range(G):
                pltpu.make_async_copy(
                    rows_v.at[b, pl.ds(t * HPAD, HPAD), :],
                    out_hbm.at[i_base + g * G + t, :, pl.ds(0, D)],
                    sem_o.at[b],
                ).wait()

        for b in range(NBUF):
            gather(b, b)

        def body(s, carry):
            g0 = s * NBUF
            for b in range(NBUF):
                wait_gather(g0 + b, b)
                writeback(g0 + b, b)
            for b in range(NBUF):
                wait_writeback(g0 + b, b)
                gather(g0 + NBUF + b, b)
            return carry

        n_passes = n_chunks // NBUF
        lax.fori_loop(0, n_passes - 1, body, 0)

        g0 = (n_passes - 1) * NBUF
        for b in range(NBUF):
            wait_gather(g0 + b, b)
            writeback(g0 + b, b)
        for b in range(NBUF):
            wait_writeback(g0 + b, b)

    return gather_kernel


def kernel(input_ids, table):
    batch, hist = input_ids.shape
    n_rows = table.shape[0]
    fill = (
        jnp.arange(batch, dtype=jnp.int32)[:, None] * (HPAD - hist)
        + jnp.arange(HPAD - hist, dtype=jnp.int32)[None, :]
    ) % n_rows
    ids56 = jnp.concatenate([input_ids, fill], axis=1)
    out_p = _make_gather(batch, 4)(ids56.reshape(-1), table * jnp.float32(1.0))
    return out_p[:, :hist, :D]


# speedup vs baseline: 1.0022x; 1.0022x over previous
import functools

import jax
import jax.numpy as jnp
from jax import lax
from jax.experimental import pallas as pl
from jax.experimental.pallas import tpu as pltpu
from jax.experimental.pallas import tpu_sc as plsc

D = 64
HIST = 50
LANES = 128
HPAD = 56
G = 4
C = G * HPAD


@functools.lru_cache(maxsize=None)
def _make_gather(BATCH: int, NBUF: int):
    info = plsc.get_sparse_core_info()
    NC, NS = info.num_cores, info.num_subcores
    NW = NC * NS
    i_per_w = BATCH // NW
    n_chunks = i_per_w // G
    assert i_per_w * NW == BATCH and n_chunks * G == i_per_w
    assert n_chunks % NBUF == 0
    n_idx = i_per_w * HPAD

    mesh = plsc.VectorSubcoreMesh(core_axis_name="c", subcore_axis_name="s")

    @functools.partial(
        pl.kernel,
        mesh=mesh,
        compiler_params=pltpu.CompilerParams(use_tc_tiling_on_sc=False),
        out_type=jax.ShapeDtypeStruct((BATCH, HPAD, LANES), jnp.float32),
        scratch_types=[
            pltpu.VMEM((n_idx,), jnp.int32),
            pltpu.VMEM((NBUF, C, D), jnp.float32),
            pltpu.SemaphoreType.DMA((NBUF,)),
            pltpu.SemaphoreType.DMA((NBUF,)),
        ],
    )
    def gather_kernel(idx_hbm, table_hbm, out_hbm, idx_v, rows_v, sem_g, sem_o):
        wid = lax.axis_index("s") * NC + lax.axis_index("c")
        i_base = wid * i_per_w

        pltpu.sync_copy(idx_hbm.at[pl.ds(i_base * HPAD, n_idx)], idx_v)

        def gather(g, b):
            pltpu.async_copy(
                table_hbm.at[idx_v.at[pl.ds(g * C, C)]],
                rows_v.at[b],
                sem_g.at[b],
            )

        def wait_gather(g, b):
            pltpu.make_async_copy(
                table_hbm.at[idx_v.at[pl.ds(g * C, C)]],
                rows_v.at[b],
                sem_g.at[b],
            ).wait()

        def writeback(g, b):
            for t in range(G):
                pltpu.async_copy(
                    rows_v.at[b, pl.ds(t * HPAD, HPAD), :],
                    out_hbm.at[i_base + g * G + t, :, pl.ds(0, D)],
                    sem_o.at[b],
                )

        def wait_writeback(g, b):
            for t in range(G):
                pltpu.make_async_copy(
                    rows_v.at[b, pl.ds(t * HPAD, HPAD), :],
                    out_hbm.at[i_base + g * G + t, :, pl.ds(0, D)],
                    sem_o.at[b],
                ).wait()

        for b in range(NBUF):
            gather(b, b)

        def body(s, carry):
            g0 = s * NBUF
            for b in range(NBUF):
                wait_gather(g0 + b, b)
                writeback(g0 + b, b)
            for b in range(NBUF):
                wait_writeback(g0 + b, b)
                gather(g0 + NBUF + b, b)
            return carry

        n_passes = n_chunks // NBUF
        lax.fori_loop(0, n_passes - 1, body, 0)

        g0 = (n_passes - 1) * NBUF
        for b in range(NBUF):
            wait_gather(g0 + b, b)
            writeback(g0 + b, b)
        for b in range(NBUF):
            wait_writeback(g0 + b, b)

    return gather_kernel


def kernel(input_ids, table):
    batch, hist = input_ids.shape
    n_rows = table.shape[0]
    fill = (
        jnp.arange(batch, dtype=jnp.int32)[:, None] * (HPAD - hist)
        + jnp.arange(HPAD - hist, dtype=jnp.int32)[None, :]
    ) % n_rows
    ids56 = jnp.concatenate([input_ids, fill], axis=1)
    out_p = _make_gather(batch, 4)(ids56.reshape(-1), table)
    return out_p[:, :hist, :D]
